# Initial kernel scaffold; baseline (speedup 1.0000x reference)
import jax
import jax.numpy as jnp
from jax.experimental import pallas as pl


def kernel(cos_theta, targets, t):
    raise NotImplementedError("write your pallas kernel here")


# speedup vs baseline: 1.0261x; 1.0261x over previous
import functools
import math

import jax
import jax.numpy as jnp
from jax import lax
from jax.experimental import pallas as pl
from jax.experimental.pallas import tpu as pltpu
from jax.experimental.pallas import tpu_sc as plsc

SCALE_C = 64.0
MARGIN_C = 0.5

_SC_CORES = 2
_SC_SUBCORES = 16
_SLIVER = 128


def _sc_gather_slivers(table, idx, b):
    n_workers = _SC_CORES * _SC_SUBCORES
    b_per_w = b // n_workers
    mesh = plsc.VectorSubcoreMesh(core_axis_name="c", subcore_axis_name="s")

    @functools.partial(
        pl.kernel,
        mesh=mesh,
        out_type=jax.ShapeDtypeStruct((b, _SLIVER), jnp.float32),
        scratch_types=[
            pltpu.VMEM((b_per_w,), jnp.int32),
            pltpu.VMEM((b_per_w, _SLIVER), jnp.float32),
            pltpu.SemaphoreType.DMA,
        ],
    )
    def gather_kernel(table_hbm, idx_hbm, out_hbm, idx_v, rows_v, sem):
        wid = lax.axis_index("s") * _SC_CORES + lax.axis_index("c")
        base = wid * b_per_w
        pltpu.sync_copy(idx_hbm.at[pl.ds(base, b_per_w)], idx_v)
        pltpu.async_copy(table_hbm.at[idx_v], rows_v, sem).wait()
        pltpu.sync_copy(rows_v, out_hbm.at[pl.ds(base, b_per_w)])

    return gather_kernel(table, idx)


def _tc_body(cos_m, sin_m, threshold, mm, inv_b, br, bc,
             x_ref, tlf_ref, mskf_ref, tgt_ref, t_ref,
             o_ref, ctm_s, ftl_s, tnew_s):
    i = pl.program_id(0)
    j = pl.program_id(1)

    @pl.when(jnp.logical_and(i == 0, j == 0))
    def _prologue():
        tlf = jnp.clip(tlf_ref[...], -1.0, 1.0) * mskf_ref[...]
        tl = jnp.sum(tlf, axis=1, keepdims=True)
        tnew_s[0, 0] = jnp.sum(tl) * (0.01 * inv_b) + 0.99 * t_ref[0, 0]
        sin_theta = jnp.sqrt(jnp.maximum(1.0 - tl * tl, 0.0))
        ctm = tl * cos_m - sin_theta * sin_m
        ctm_s[...] = ctm
        ftl_s[...] = jnp.where(tl > threshold, ctm, tl - mm) * SCALE_C

    t_new = tnew_s[0, 0]
    ctm = ctm_s[pl.ds(i * br, br), :]
    ftl = ftl_s[pl.ds(i * br, br), :]
    ct = jnp.clip(x_ref[...], -1.0, 1.0)
    s = ct * SCALE_C
    r = jnp.where(ct > ctm, s * (t_new + ct), s)
    col = lax.broadcasted_iota(jnp.int32, r.shape, 1) + j * bc
    o_ref[...] = jnp.where(col == tgt_ref[...], ftl, r)


def kernel(cos_theta, targets, t):
    b, c = cos_theta.shape
    cos_m = math.cos(MARGIN_C)
    sin_m = math.sin(MARGIN_C)
    threshold = math.cos(math.pi - MARGIN_C)
    mm = math.sin(math.pi - MARGIN_C) * MARGIN_C

    t32 = targets.astype(jnp.int32)
    flat = jnp.arange(b, dtype=jnp.int32) * c + t32
    sliver_idx = flat // _SLIVER
    lane = flat % _SLIVER
    mskf = (lane[:, None] == jnp.arange(_SLIVER, dtype=jnp.int32)[None, :]
            ).astype(jnp.float32)

    table = cos_theta.reshape(b * c // _SLIVER, _SLIVER)
    tlf = _sc_gather_slivers(table, sliver_idx, b)

    tgt2 = t32[:, None]
    t2 = t.reshape(1, 1).astype(jnp.float32)

    br, bc = 256, 4096
    grid = (b // br, pl.cdiv(c, bc))
    body = functools.partial(_tc_body, cos_m, sin_m, threshold, mm, 1.0 / b,
                             br, bc)
    out = pl.pallas_call(
        body,
        grid=grid,
        in_specs=[
            pl.BlockSpec((br, bc), lambda i, j: (i, j)),
            pl.BlockSpec((b, _SLIVER), lambda i, j: (0, 0)),
            pl.BlockSpec((b, _SLIVER), lambda i, j: (0, 0)),
            pl.BlockSpec((br, 1), lambda i, j: (i, 0)),
            pl.BlockSpec((1, 1), lambda i, j: (0, 0)),
        ],
        out_specs=pl.BlockSpec((br, bc), lambda i, j: (i, j)),
        out_shape=jax.ShapeDtypeStruct((b, c), jnp.float32),
        scratch_shapes=[
            pltpu.VMEM((b, 1), jnp.float32),
            pltpu.VMEM((b, 1), jnp.float32),
            pltpu.SMEM((1, 1), jnp.float32),
        ],
    )(cos_theta, tlf, mskf, tgt2, t2)
    return out


# speedup vs baseline: 1.4250x; 1.3888x over previous
import functools
import math

import jax
import jax.numpy as jnp
from jax import lax
from jax.experimental import pallas as pl
from jax.experimental.pallas import tpu as pltpu
from jax.experimental.pallas import tpu_sc as plsc

SCALE_C = 64.0
MARGIN_C = 0.5

_SC_CORES = 2
_SLIVER = 128
_ROWBLK = 8


def _sc_gather_slivers(cos_theta, c0, b):
    b_per_core = b // _SC_CORES
    mesh = plsc.ScalarSubcoreMesh(axis_name="core", num_cores=_SC_CORES)

    @functools.partial(
        pl.kernel,
        mesh=mesh,
        out_type=jax.ShapeDtypeStruct((b, _ROWBLK, _SLIVER), jnp.float32),
        scratch_types=[
            pltpu.SMEM((b_per_core,), jnp.int32),
            pltpu.SemaphoreType.DMA,
            pltpu.SemaphoreType.DMA,
        ],
    )
    def gather_kernel(cos_hbm, c0_hbm, out_hbm, c0_s, sem_in, sem):
        core = lax.axis_index("core")
        base = core * b_per_core
        pltpu.async_copy(c0_hbm.at[pl.ds(base, b_per_core)], c0_s,
                         sem_in).wait()

        @pl.loop(0, b_per_core)
        def _fire(k):
            row = base + k
            row0 = pl.multiple_of((row // _ROWBLK) * _ROWBLK, _ROWBLK)
            start = pl.multiple_of(c0_s[k], _SLIVER)
            pltpu.async_copy(
                cos_hbm.at[pl.ds(row0, _ROWBLK), pl.ds(start, _SLIVER)],
                out_hbm.at[row], sem)

        pltpu.make_async_copy(out_hbm.at[pl.ds(base, b_per_core)],
                              out_hbm.at[pl.ds(base, b_per_core)],
                              sem).wait()

    return gather_kernel(cos_theta, c0)


def _tc_body(cos_m, sin_m, threshold, mm, inv_b, br, bc,
             x_ref, tlf_ref, sel_ref, tgt_ref, t_ref,
             o_ref, ctm_s, ftl_s, tnew_s):
    i = pl.program_id(0)
    j = pl.program_id(1)

    @pl.when(jnp.logical_and(i == 0, j == 0))
    def _prologue():
        tlf = jnp.clip(tlf_ref[...], -1.0, 1.0)
        pos = lax.broadcasted_iota(jnp.int32, tlf.shape, 1)
        m = (pos == sel_ref[...]).astype(jnp.float32)
        tl = jnp.sum(tlf * m, axis=1, keepdims=True)
        tnew_s[0, 0] = jnp.sum(tl) * (0.01 * inv_b) + 0.99 * t_ref[0, 0]
        sin_theta = jnp.sqrt(jnp.maximum(1.0 - tl * tl, 0.0))
        ctm = tl * cos_m - sin_theta * sin_m
        ctm_s[...] = ctm
        ftl_s[...] = jnp.where(tl > threshold, ctm, tl - mm) * SCALE_C

    t_new = tnew_s[0, 0]
    ctm = ctm_s[pl.ds(i * br, br), :]
    ftl = ftl_s[pl.ds(i * br, br), :]
    ct = jnp.clip(x_ref[...], -1.0, 1.0)
    s = ct * SCALE_C
    r = jnp.where(ct > ctm, s * (t_new + ct), s)
    col = lax.broadcasted_iota(jnp.int32, r.shape, 1) + j * bc
    o_ref[...] = jnp.where(col == tgt_ref[...], ftl, r)


def kernel(cos_theta, targets, t):
    b, c = cos_theta.shape
    cos_m = math.cos(MARGIN_C)
    sin_m = math.sin(MARGIN_C)
    threshold = math.cos(math.pi - MARGIN_C)
    mm = math.sin(math.pi - MARGIN_C) * MARGIN_C

    t32 = targets.astype(jnp.int32)
    lane = t32 % _SLIVER
    c0 = t32 - lane

    sliver3 = _sc_gather_slivers(cos_theta, c0, b)
    tlf = sliver3.reshape(b, _ROWBLK * _SLIVER)
    sel = ((jnp.arange(b, dtype=jnp.int32) % _ROWBLK) * _SLIVER
           + lane)[:, None]
    tgt2 = t32[:, None]
    t2 = t.reshape(1, 1).astype(jnp.float32)

    br, bc = 256, 4096
    grid = (b // br, pl.cdiv(c, bc))
    body = functools.partial(_tc_body, cos_m, sin_m, threshold, mm, 1.0 / b,
                             br, bc)
    out = pl.pallas_call(
        body,
        grid=grid,
        in_specs=[
            pl.BlockSpec((br, bc), lambda i, j: (i, j)),
            pl.BlockSpec(tlf.shape, lambda i, j: (0, 0)),
            pl.BlockSpec((b, 1), lambda i, j: (0, 0)),
            pl.BlockSpec((br, 1), lambda i, j: (i, 0)),
            pl.BlockSpec((1, 1), lambda i, j: (0, 0)),
        ],
        out_specs=pl.BlockSpec((br, bc), lambda i, j: (i, j)),
        out_shape=jax.ShapeDtypeStruct((b, c), jnp.float32),
        scratch_shapes=[
            pltpu.VMEM((b, 1), jnp.float32),
            pltpu.VMEM((b, 1), jnp.float32),
            pltpu.SMEM((1, 1), jnp.float32),
        ],
    )(cos_theta, tlf, sel, tgt2, t2)
    return out
